# baseline (speedup 1.0000x reference)
import functools

import jax
import jax.numpy as jnp
from jax import lax
from jax.experimental import pallas as pl
from jax.experimental.pallas import tpu as pltpu
from jax.experimental.pallas import tpu_sc as plsc

N = 10000
NP = 10240
D = 128
E = 320000
NC = 2
NS = 16
NW = NC * NS
EPT = E // NW
EPP = 10240
CHE = 128
NCH = EPP // CHE
RPT = NP // NS

RB = 2000
NB = N // RB


NPR = NP // D


def _mesh():
    return plsc.VectorSubcoreMesh(core_axis_name="c", subcore_axis_name="s")


def _deg_call(dst, zrd):
    @functools.partial(
        pl.kernel,
        out_type=jax.ShapeDtypeStruct((NC * NPR, D), jnp.float32),
        mesh=_mesh(),
        compiler_params=pltpu.CompilerParams(needs_layout_passes=False),
        scratch_types=[
            pltpu.VMEM((NCH, CHE), jnp.int32),
            pltpu.VMEM((NPR, D), jnp.float32),
            pltpu.VMEM((NPR,), jnp.int32),
            pltpu.VMEM_SHARED((NPR, D), jnp.float32),
        ],
    )
    def deg_kernel(dst_hbm, z_hbm, out_hbm, didx, hist, rix, shacc):
        c = lax.axis_index("c")
        s = lax.axis_index("s")
        t = c * NS + s
        iota = lax.iota(jnp.int32, 16)
        zero16 = jnp.zeros((16,), jnp.float32)

        @pl.when(s < 10)
        def _():
            pltpu.sync_copy(z_hbm.at[pl.ds(s * 8, 8)], shacc.at[pl.ds(s * 8, 8)])

        pltpu.sync_copy(dst_hbm.at[t], didx)

        for k in range(NPR // 16):
            rix[pl.ds(k * 16, 16)] = iota + k * 16

        def zbody(j, carry):
            for k in range(8):
                hist[j, pl.ds(k * 16, 16)] = zero16
            return carry

        lax.fori_loop(0, NPR, zbody, 0)

        def body(j, carry):
            for k in range(CHE // 16):
                v = didx[j, pl.ds(k * 16, 16)]
                cnt, last = plsc.scan_count(v)
                vhi = lax.shift_right_logical(v, 7)
                vlo = lax.bitwise_and(v, 127)
                plsc.addupdate_scatter(hist, [vhi, vlo],
                                       cnt.astype(jnp.float32), mask=last)
            return carry

        lax.fori_loop(0, NCH, body, 0)
        plsc.subcore_barrier()
        pltpu.sync_copy(hist, shacc.at[rix], add=True)
        plsc.subcore_barrier()

        @pl.when(s < 10)
        def _():
            pltpu.sync_copy(shacc.at[pl.ds(s * 8, 8)],
                            out_hbm.at[pl.ds(c * NPR + s * 8, 8)])

    return deg_kernel(dst, zrd)


def _edge_call(g, s3, d3, znd):
    @functools.partial(
        pl.kernel,
        out_type=jax.ShapeDtypeStruct((NC * NP, D), jnp.float32),
        mesh=_mesh(),
        compiler_params=pltpu.CompilerParams(needs_layout_passes=False),
        scratch_types=[
            pltpu.VMEM((NCH, CHE), jnp.int32),
            pltpu.VMEM((NCH, CHE), jnp.int32),
            pltpu.VMEM((CHE, D), jnp.float32),
            pltpu.SemaphoreType.DMA,
            pltpu.VMEM_SHARED((NP, D), jnp.float32),
        ],
    )
    def edge_kernel(g_hbm, s3_hbm, d3_hbm, z_hbm, out_hbm,
                    sidx, didx, rows, sem, acc):
        c = lax.axis_index("c")
        s = lax.axis_index("s")
        t = c * NS + s
        pltpu.sync_copy(s3_hbm.at[t], sidx)
        pltpu.sync_copy(d3_hbm.at[t], didx)
        pltpu.sync_copy(z_hbm.at[pl.ds(s * RPT, RPT)], acc.at[pl.ds(s * RPT, RPT)])
        plsc.subcore_barrier()

        def body(j, carry):
            pltpu.async_copy(g_hbm.at[sidx.at[j]], rows, sem).wait()
            pltpu.sync_copy(rows, acc.at[didx.at[j]], add=True)
            return carry

        lax.fori_loop(0, NCH, body, 0)
        plsc.subcore_barrier()
        pltpu.sync_copy(acc.at[pl.ds(s * RPT, RPT)],
                        out_hbm.at[pl.ds(c * NP + s * RPT, RPT)])

    return edge_kernel(g, s3, d3, znd)


def _mm(a, b):
    return lax.dot_general(a, b, (((1,), (0,)), ((), ())),
                           precision=lax.Precision.HIGHEST,
                           preferred_element_type=jnp.float32)


def _tc_prep(x, W1, dinv_col):
    def body(x_ref, w_ref, dv_ref, h_ref, g_ref):
        dinv = dv_ref[...]
        h = _mm(x_ref[...], w_ref[...])
        h_ref[...] = h
        g_ref[...] = h * dinv

    return pl.pallas_call(
        body,
        grid=(NB,),
        in_specs=[
            pl.BlockSpec((RB, D), lambda i: (i, 0)),
            pl.BlockSpec((D, D), lambda i: (0, 0)),
            pl.BlockSpec((RB, 1), lambda i: (i, 0)),
        ],
        out_specs=[pl.BlockSpec((RB, D), lambda i: (i, 0))] * 2,
        out_shape=[jax.ShapeDtypeStruct((N, D), jnp.float32)] * 2,
    )(x, W1, dinv_col)


def _tc_mid(accp, h1, dinv_col, b1r, W2):
    def body(aa_ref, ab_ref, h1_ref, dv_ref, b_ref, w_ref, h2_ref, g2_ref):
        dinv = dv_ref[...]
        agg = aa_ref[0] + ab_ref[0]
        o1 = jnp.maximum(
            dinv * agg + dinv * dinv * h1_ref[...] + b_ref[...], 0.0)
        h2 = _mm(o1, w_ref[...])
        h2_ref[...] = h2
        g2_ref[...] = h2 * dinv

    return pl.pallas_call(
        body,
        grid=(NB,),
        in_specs=[
            pl.BlockSpec((1, RB, D), lambda i: (0, i, 0)),
            pl.BlockSpec((1, RB, D), lambda i: (1, i, 0)),
            pl.BlockSpec((RB, D), lambda i: (i, 0)),
            pl.BlockSpec((RB, 1), lambda i: (i, 0)),
            pl.BlockSpec((1, D), lambda i: (0, 0)),
            pl.BlockSpec((D, D), lambda i: (0, 0)),
        ],
        out_specs=[pl.BlockSpec((RB, D), lambda i: (i, 0))] * 2,
        out_shape=[jax.ShapeDtypeStruct((N, D), jnp.float32)] * 2,
    )(accp, accp, h1, dinv_col, b1r, W2)


def _tc_final(accp, h2, dinv_col, b2r):
    def body(aa_ref, ab_ref, h2_ref, dv_ref, b_ref, out_ref):
        dinv = dv_ref[...]
        agg = aa_ref[0] + ab_ref[0]
        out_ref[...] = dinv * agg + dinv * dinv * h2_ref[...] + b_ref[...]

    return pl.pallas_call(
        body,
        grid=(NB,),
        in_specs=[
            pl.BlockSpec((1, RB, D), lambda i: (0, i, 0)),
            pl.BlockSpec((1, RB, D), lambda i: (1, i, 0)),
            pl.BlockSpec((RB, D), lambda i: (i, 0)),
            pl.BlockSpec((RB, 1), lambda i: (i, 0)),
            pl.BlockSpec((1, D), lambda i: (0, 0)),
        ],
        out_specs=pl.BlockSpec((RB, D), lambda i: (i, 0)),
        out_shape=jax.ShapeDtypeStruct((N, D), jnp.float32),
    )(accp, accp, h2, dinv_col, b2r)


def kernel(x, edge_index, W1, b1, W2, b2):
    ei = edge_index.astype(jnp.int32)
    s3 = jnp.concatenate(
        [ei[0].reshape(NW, EPT),
         jnp.zeros((NW, EPP - EPT), jnp.int32)], axis=1).reshape(NW, NCH, CHE)
    d3 = jnp.concatenate(
        [ei[1].reshape(NW, EPT),
         jnp.full((NW, EPP - EPT), NP - 1, jnp.int32)],
        axis=1).reshape(NW, NCH, CHE)
    znd = jnp.zeros((NP, D), jnp.float32)

    degp = _deg_call(d3, znd[:NPR]).reshape(NC, NP)
    dinv_col = lax.rsqrt(degp[0, :N] + degp[1, :N] + 1.0).reshape(N, 1)

    h1, g1 = _tc_prep(x, W1, dinv_col)
    acc1 = _edge_call(g1, s3, d3, znd).reshape(NC, NP, D)
    h2, g2 = _tc_mid(acc1, h1, dinv_col, b1.reshape(1, D), W2)
    acc2 = _edge_call(g2, s3, d3, znd).reshape(NC, NP, D)
    return _tc_final(acc2, h2, dinv_col, b2.reshape(1, D))


# speedup vs baseline: 2.2123x; 2.2123x over previous
import functools

import jax
import jax.numpy as jnp
from jax import lax
from jax.experimental import pallas as pl
from jax.experimental.pallas import tpu as pltpu
from jax.experimental.pallas import tpu_sc as plsc

N = 10000
NP = 10240
D = 128
E = 320000
NC = 2
NS = 16
NW = NC * NS
EPT = E // NW
CH = 80
NCHUNK = EPT // CH
EPP = 10240
CHE = 128
NCH = EPP // CHE
RPT = NP // NS

RB = 2000
NB = N // RB


NPR = NP // D


def _mesh():
    return plsc.VectorSubcoreMesh(core_axis_name="c", subcore_axis_name="s")


def _deg_call(dst, zrd):
    @functools.partial(
        pl.kernel,
        out_type=jax.ShapeDtypeStruct((NC * NPR, D), jnp.float32),
        mesh=_mesh(),
        compiler_params=pltpu.CompilerParams(needs_layout_passes=False),
        scratch_types=[
            pltpu.VMEM((NCH, CHE), jnp.int32),
            pltpu.VMEM((NPR, D), jnp.float32),
            pltpu.VMEM((NPR,), jnp.int32),
            pltpu.VMEM_SHARED((NPR, D), jnp.float32),
        ],
    )
    def deg_kernel(dst_hbm, z_hbm, out_hbm, didx, hist, rix, shacc):
        c = lax.axis_index("c")
        s = lax.axis_index("s")
        t = c * NS + s
        iota = lax.iota(jnp.int32, 16)
        zero16 = jnp.zeros((16,), jnp.float32)

        @pl.when(s < 10)
        def _():
            pltpu.sync_copy(z_hbm.at[pl.ds(s * 8, 8)], shacc.at[pl.ds(s * 8, 8)])

        pltpu.sync_copy(dst_hbm.at[t], didx)

        for k in range(NPR // 16):
            rix[pl.ds(k * 16, 16)] = iota + k * 16

        def zbody(j, carry):
            for k in range(8):
                hist[j, pl.ds(k * 16, 16)] = zero16
            return carry

        lax.fori_loop(0, NPR, zbody, 0)

        def body(j, carry):
            for k in range(CHE // 16):
                v = didx[j, pl.ds(k * 16, 16)]
                cnt, last = plsc.scan_count(v)
                vhi = lax.shift_right_logical(v, 7)
                vlo = lax.bitwise_and(v, 127)
                plsc.addupdate_scatter(hist, [vhi, vlo],
                                       cnt.astype(jnp.float32), mask=last)
            return carry

        lax.fori_loop(0, NCH, body, 0)
        plsc.subcore_barrier()
        pltpu.sync_copy(hist, shacc.at[rix], add=True)
        plsc.subcore_barrier()

        @pl.when(s < 10)
        def _():
            pltpu.sync_copy(shacc.at[pl.ds(s * 8, 8)],
                            out_hbm.at[pl.ds(c * NPR + s * 8, 8)])

    return deg_kernel(dst, zrd)


def _edge_call(g, src, dst, znd):
    @functools.partial(
        pl.kernel,
        out_type=jax.ShapeDtypeStruct((NC * NP, D), jnp.float32),
        mesh=_mesh(),
        compiler_params=pltpu.CompilerParams(needs_layout_passes=False),
        scratch_types=[
            pltpu.VMEM((CH,), jnp.int32),
            pltpu.VMEM((CH,), jnp.int32),
            pltpu.VMEM((CH,), jnp.int32),
            pltpu.VMEM((CH,), jnp.int32),
            pltpu.VMEM((CH, D), jnp.float32),
            pltpu.VMEM((CH, D), jnp.float32),
            pltpu.SemaphoreType.DMA,
            pltpu.SemaphoreType.DMA,
            pltpu.VMEM_SHARED((NP, D), jnp.float32),
        ],
    )
    def edge_kernel(g_hbm, src_hbm, dst_hbm, z_hbm, out_hbm,
                    sidx0, sidx1, didx0, didx1, rows0, rows1,
                    sem0, sem1, acc):
        c = lax.axis_index("c")
        s = lax.axis_index("s")
        t = c * NS + s
        sidx = (sidx0, sidx1)
        didx = (didx0, didx1)
        rows = (rows0, rows1)
        sems = (sem0, sem1)
        pltpu.sync_copy(z_hbm.at[pl.ds(s * RPT, RPT)], acc.at[pl.ds(s * RPT, RPT)])
        plsc.subcore_barrier()
        base = t * EPT

        def load_issue(j, b):
            off = pl.multiple_of(base + j * CH, 8)
            pltpu.sync_copy(src_hbm.at[pl.ds(off, CH)], sidx[b])
            pltpu.sync_copy(dst_hbm.at[pl.ds(off, CH)], didx[b])
            pltpu.async_copy(g_hbm.at[sidx[b]], rows[b], sems[b])

        load_issue(0, 0)
        load_issue(1, 1)

        def body(jj, carry):
            for b in range(2):
                j = jj * 2 + b
                pltpu.make_async_copy(g_hbm.at[sidx[b]], rows[b],
                                      sems[b]).wait()
                pltpu.sync_copy(rows[b], acc.at[didx[b]], add=True)

                @pl.when(j + 2 < NCHUNK)
                def _():
                    load_issue(j + 2, b)
            return carry

        lax.fori_loop(0, NCHUNK // 2, body, 0)
        pltpu.make_async_copy(g_hbm.at[sidx0], rows0, sem0).wait()
        pltpu.sync_copy(rows0, acc.at[didx0], add=True)
        plsc.subcore_barrier()
        pltpu.sync_copy(acc.at[pl.ds(s * RPT, RPT)],
                        out_hbm.at[pl.ds(c * NP + s * RPT, RPT)])

    return edge_kernel(g, src, dst, znd)


def _mm(a, b):
    return lax.dot_general(a, b, (((1,), (0,)), ((), ())),
                           precision=lax.Precision.HIGHEST,
                           preferred_element_type=jnp.float32)


def _tc_prep(x, W1, dinv_col):
    def body(x_ref, w_ref, dv_ref, h_ref, g_ref):
        dinv = dv_ref[...]
        h = _mm(x_ref[...], w_ref[...])
        h_ref[...] = h
        g_ref[...] = h * dinv

    return pl.pallas_call(
        body,
        grid=(NB,),
        in_specs=[
            pl.BlockSpec((RB, D), lambda i: (i, 0)),
            pl.BlockSpec((D, D), lambda i: (0, 0)),
            pl.BlockSpec((RB, 1), lambda i: (i, 0)),
        ],
        out_specs=[pl.BlockSpec((RB, D), lambda i: (i, 0))] * 2,
        out_shape=[jax.ShapeDtypeStruct((N, D), jnp.float32)] * 2,
    )(x, W1, dinv_col)


def _tc_mid(accp, h1, dinv_col, b1r, W2):
    def body(aa_ref, ab_ref, h1_ref, dv_ref, b_ref, w_ref, h2_ref, g2_ref):
        dinv = dv_ref[...]
        agg = aa_ref[0] + ab_ref[0]
        o1 = jnp.maximum(
            dinv * agg + dinv * dinv * h1_ref[...] + b_ref[...], 0.0)
        h2 = _mm(o1, w_ref[...])
        h2_ref[...] = h2
        g2_ref[...] = h2 * dinv

    return pl.pallas_call(
        body,
        grid=(NB,),
        in_specs=[
            pl.BlockSpec((1, RB, D), lambda i: (0, i, 0)),
            pl.BlockSpec((1, RB, D), lambda i: (1, i, 0)),
            pl.BlockSpec((RB, D), lambda i: (i, 0)),
            pl.BlockSpec((RB, 1), lambda i: (i, 0)),
            pl.BlockSpec((1, D), lambda i: (0, 0)),
            pl.BlockSpec((D, D), lambda i: (0, 0)),
        ],
        out_specs=[pl.BlockSpec((RB, D), lambda i: (i, 0))] * 2,
        out_shape=[jax.ShapeDtypeStruct((N, D), jnp.float32)] * 2,
    )(accp, accp, h1, dinv_col, b1r, W2)


def _tc_final(accp, h2, dinv_col, b2r):
    def body(aa_ref, ab_ref, h2_ref, dv_ref, b_ref, out_ref):
        dinv = dv_ref[...]
        agg = aa_ref[0] + ab_ref[0]
        out_ref[...] = dinv * agg + dinv * dinv * h2_ref[...] + b_ref[...]

    return pl.pallas_call(
        body,
        grid=(NB,),
        in_specs=[
            pl.BlockSpec((1, RB, D), lambda i: (0, i, 0)),
            pl.BlockSpec((1, RB, D), lambda i: (1, i, 0)),
            pl.BlockSpec((RB, D), lambda i: (i, 0)),
            pl.BlockSpec((RB, 1), lambda i: (i, 0)),
            pl.BlockSpec((1, D), lambda i: (0, 0)),
        ],
        out_specs=pl.BlockSpec((RB, D), lambda i: (i, 0)),
        out_shape=jax.ShapeDtypeStruct((N, D), jnp.float32),
    )(accp, accp, h2, dinv_col, b2r)


def kernel(x, edge_index, W1, b1, W2, b2):
    ei = edge_index.astype(jnp.int32)
    d3 = jnp.concatenate(
        [ei[1].reshape(NW, EPT),
         jnp.full((NW, EPP - EPT), NP - 1, jnp.int32)],
        axis=1).reshape(NW, NCH, CHE)
    src = ei[0]
    dst = ei[1]
    znd = jnp.zeros((NP, D), jnp.float32)

    degp = _deg_call(d3, znd[:NPR]).reshape(NC, NP)
    dinv_col = lax.rsqrt(degp[0, :N] + degp[1, :N] + 1.0).reshape(N, 1)

    h1, g1 = _tc_prep(x, W1, dinv_col)
    acc1 = _edge_call(g1, src, dst, znd).reshape(NC, NP, D)
    h2, g2 = _tc_mid(acc1, h1, dinv_col, b1.reshape(1, D), W2)
    acc2 = _edge_call(g2, src, dst, znd).reshape(NC, NP, D)
    return _tc_final(acc2, h2, dinv_col, b2.reshape(1, D))


# speedup vs baseline: 2.9888x; 1.3510x over previous
import functools

import jax
import jax.numpy as jnp
from jax import lax
from jax.experimental import pallas as pl
from jax.experimental.pallas import tpu as pltpu
from jax.experimental.pallas import tpu_sc as plsc

N = 10000
NP = 10240
D = 128
E = 320000
NC = 2
NS = 16
NW = NC * NS
EPT = E // NW
CH = 80
NCHUNK = EPT // CH
EPP = 10240
CHE = 128
NCH = EPP // CHE
RPT = NP // NS

RB = 2000
NB = N // RB


NPR = NP // D


def _mesh():
    return plsc.VectorSubcoreMesh(core_axis_name="c", subcore_axis_name="s")


def _deg_call(dst, zrd):
    @functools.partial(
        pl.kernel,
        out_type=jax.ShapeDtypeStruct((NC * NPR, D), jnp.float32),
        mesh=_mesh(),
        compiler_params=pltpu.CompilerParams(needs_layout_passes=False),
        scratch_types=[
            pltpu.VMEM((NCH, CHE), jnp.int32),
            pltpu.VMEM((NPR, D), jnp.float32),
            pltpu.VMEM((NPR,), jnp.int32),
            pltpu.VMEM_SHARED((NPR, D), jnp.float32),
        ],
    )
    def deg_kernel(dst_hbm, z_hbm, out_hbm, didx, hist, rix, shacc):
        c = lax.axis_index("c")
        s = lax.axis_index("s")
        t = c * NS + s
        iota = lax.iota(jnp.int32, 16)
        zero16 = jnp.zeros((16,), jnp.float32)

        @pl.when(s < 10)
        def _():
            pltpu.sync_copy(z_hbm.at[pl.ds(s * 8, 8)], shacc.at[pl.ds(s * 8, 8)])

        pltpu.sync_copy(dst_hbm.at[t], didx)

        for k in range(NPR // 16):
            rix[pl.ds(k * 16, 16)] = iota + k * 16

        def zbody(j, carry):
            for k in range(8):
                hist[j, pl.ds(k * 16, 16)] = zero16
            return carry

        lax.fori_loop(0, NPR, zbody, 0)

        def body(j, carry):
            for k in range(CHE // 16):
                v = didx[j, pl.ds(k * 16, 16)]
                cnt, last = plsc.scan_count(v)
                vhi = lax.shift_right_logical(v, 7)
                vlo = lax.bitwise_and(v, 127)
                plsc.addupdate_scatter(hist, [vhi, vlo],
                                       cnt.astype(jnp.float32), mask=last)
            return carry

        lax.fori_loop(0, NCH, body, 0)
        plsc.subcore_barrier()
        pltpu.sync_copy(hist, shacc.at[rix], add=True)
        plsc.subcore_barrier()

        @pl.when(s < 10)
        def _():
            pltpu.sync_copy(shacc.at[pl.ds(s * 8, 8)],
                            out_hbm.at[pl.ds(c * NPR + s * 8, 8)])

    return deg_kernel(dst, zrd)


def _edge_call(g, src, dst, znd):
    @functools.partial(
        pl.kernel,
        out_type=jax.ShapeDtypeStruct((NC * NP, D), jnp.float32),
        mesh=_mesh(),
        compiler_params=pltpu.CompilerParams(needs_layout_passes=False),
        scratch_types=[
            [pltpu.VMEM((CH,), jnp.int32)] * 4,
            [pltpu.VMEM((CH,), jnp.int32)] * 4,
            [pltpu.VMEM((CH, D), jnp.float32)] * 2,
            [pltpu.SemaphoreType.DMA] * 2,
            [pltpu.SemaphoreType.DMA] * 4,
            pltpu.VMEM_SHARED((NP, D), jnp.float32),
        ],
    )
    def edge_kernel(g_hbm, src_hbm, dst_hbm, z_hbm, out_hbm,
                    sidx, didx, rows, gsems, isems, acc):
        c = lax.axis_index("c")
        s = lax.axis_index("s")
        t = c * NS + s
        pltpu.sync_copy(z_hbm.at[pl.ds(s * RPT, RPT)], acc.at[pl.ds(s * RPT, RPT)])
        plsc.subcore_barrier()
        base = t * EPT

        def idx_descs(j, bi):
            off = pl.multiple_of(base + j * CH, 8)
            return (
                pltpu.make_async_copy(src_hbm.at[pl.ds(off, CH)], sidx[bi],
                                      isems[bi]),
                pltpu.make_async_copy(dst_hbm.at[pl.ds(off, CH)], didx[bi],
                                      isems[bi]),
            )

        for j in (0, 1):
            for d in idx_descs(j, j):
                d.start()
                d.wait()
            pltpu.async_copy(g_hbm.at[sidx[j]], rows[j], gsems[j])

        def step(j, br, bi, ni):
            pltpu.make_async_copy(g_hbm.at[sidx[bi]], rows[br],
                                  gsems[br]).wait()

            @pl.when(j + 2 < NCHUNK)
            def _():
                for d in idx_descs(j + 2, ni):
                    d.start()

            pltpu.sync_copy(rows[br], acc.at[didx[bi]], add=True)

            @pl.when(j + 2 < NCHUNK)
            def _():
                for d in idx_descs(j + 2, ni):
                    d.wait()
                pltpu.async_copy(g_hbm.at[sidx[ni]], rows[br], gsems[br])

        def body(jj, carry):
            for b in range(4):
                j = jj * 4 + b
                step(j, b % 2, b, (b + 2) % 4)
            return carry

        lax.fori_loop(0, (NCHUNK - 1) // 4, body, 0)
        step(NCHUNK - 1, (NCHUNK - 1) % 2, (NCHUNK - 1) % 4, 0)
        plsc.subcore_barrier()
        pltpu.sync_copy(acc.at[pl.ds(s * RPT, RPT)],
                        out_hbm.at[pl.ds(c * NP + s * RPT, RPT)])

    return edge_kernel(g, src, dst, znd)


def _mm(a, b):
    return lax.dot_general(a, b, (((1,), (0,)), ((), ())),
                           precision=lax.Precision.HIGHEST,
                           preferred_element_type=jnp.float32)


def _tc_prep(x, W1, dinv_col):
    def body(x_ref, w_ref, dv_ref, h_ref, g_ref):
        dinv = dv_ref[...]
        h = _mm(x_ref[...], w_ref[...])
        h_ref[...] = h
        g_ref[...] = h * dinv

    return pl.pallas_call(
        body,
        grid=(NB,),
        in_specs=[
            pl.BlockSpec((RB, D), lambda i: (i, 0)),
            pl.BlockSpec((D, D), lambda i: (0, 0)),
            pl.BlockSpec((RB, 1), lambda i: (i, 0)),
        ],
        out_specs=[pl.BlockSpec((RB, D), lambda i: (i, 0))] * 2,
        out_shape=[jax.ShapeDtypeStruct((N, D), jnp.float32)] * 2,
    )(x, W1, dinv_col)


def _tc_mid(accp, h1, dinv_col, b1r, W2):
    def body(aa_ref, ab_ref, h1_ref, dv_ref, b_ref, w_ref, h2_ref, g2_ref):
        dinv = dv_ref[...]
        agg = aa_ref[0] + ab_ref[0]
        o1 = jnp.maximum(
            dinv * agg + dinv * dinv * h1_ref[...] + b_ref[...], 0.0)
        h2 = _mm(o1, w_ref[...])
        h2_ref[...] = h2
        g2_ref[...] = h2 * dinv

    return pl.pallas_call(
        body,
        grid=(NB,),
        in_specs=[
            pl.BlockSpec((1, RB, D), lambda i: (0, i, 0)),
            pl.BlockSpec((1, RB, D), lambda i: (1, i, 0)),
            pl.BlockSpec((RB, D), lambda i: (i, 0)),
            pl.BlockSpec((RB, 1), lambda i: (i, 0)),
            pl.BlockSpec((1, D), lambda i: (0, 0)),
            pl.BlockSpec((D, D), lambda i: (0, 0)),
        ],
        out_specs=[pl.BlockSpec((RB, D), lambda i: (i, 0))] * 2,
        out_shape=[jax.ShapeDtypeStruct((N, D), jnp.float32)] * 2,
    )(accp, accp, h1, dinv_col, b1r, W2)


def _tc_final(accp, h2, dinv_col, b2r):
    def body(aa_ref, ab_ref, h2_ref, dv_ref, b_ref, out_ref):
        dinv = dv_ref[...]
        agg = aa_ref[0] + ab_ref[0]
        out_ref[...] = dinv * agg + dinv * dinv * h2_ref[...] + b_ref[...]

    return pl.pallas_call(
        body,
        grid=(NB,),
        in_specs=[
            pl.BlockSpec((1, RB, D), lambda i: (0, i, 0)),
            pl.BlockSpec((1, RB, D), lambda i: (1, i, 0)),
            pl.BlockSpec((RB, D), lambda i: (i, 0)),
            pl.BlockSpec((RB, 1), lambda i: (i, 0)),
            pl.BlockSpec((1, D), lambda i: (0, 0)),
        ],
        out_specs=pl.BlockSpec((RB, D), lambda i: (i, 0)),
        out_shape=jax.ShapeDtypeStruct((N, D), jnp.float32),
    )(accp, accp, h2, dinv_col, b2r)


def kernel(x, edge_index, W1, b1, W2, b2):
    ei = edge_index.astype(jnp.int32)
    d3 = jnp.concatenate(
        [ei[1].reshape(NW, EPT),
         jnp.full((NW, EPP - EPT), NP - 1, jnp.int32)],
        axis=1).reshape(NW, NCH, CHE)
    src = ei[0]
    dst = ei[1]
    znd = jnp.zeros((NP, D), jnp.float32)

    degp = _deg_call(d3, znd[:NPR]).reshape(NC, NP)
    dinv_col = lax.rsqrt(degp[0, :N] + degp[1, :N] + 1.0).reshape(N, 1)

    h1, g1 = _tc_prep(x, W1, dinv_col)
    acc1 = _edge_call(g1, src, dst, znd).reshape(NC, NP, D)
    h2, g2 = _tc_mid(acc1, h1, dinv_col, b1.reshape(1, D), W2)
    acc2 = _edge_call(g2, src, dst, znd).reshape(NC, NP, D)
    return _tc_final(acc2, h2, dinv_col, b2.reshape(1, D))
